# baseline (speedup 1.0000x reference)
import functools

import jax
import jax.numpy as jnp
from jax import lax
from jax.experimental import pallas as pl
from jax.experimental.pallas import tpu as pltpu
from jax.experimental.pallas import tpu_sc as plsc

PAD_ID = 0
B, N, L = 64, 2048, 32
LANES = 16
V = N // LANES
HALF = V // 2
ROWS_PER_W = 2


def _vsort_desc(v):
    return lax.rev(jnp.sort(v), (0,))


@functools.lru_cache(maxsize=1)
def _build_sc_sort():
    info = plsc.get_sparse_core_info()
    nc = info.num_cores

    def body(ng_hbm, bt_hbm, nll_hbm, qa_hbm, sorted_hbm,
             ng_v, bt_v, nll_v, qa_v, comb_v, sem):
        wid = lax.axis_index("s") * nc + lax.axis_index("c")
        row0 = wid * ROWS_PER_W

        copies = []
        for r in range(ROWS_PER_W):
            row = row0 + r
            copies.append(pltpu.async_copy(ng_hbm.at[row], ng_v.at[r], sem))
            copies.append(pltpu.async_copy(bt_hbm.at[row], bt_v.at[r], sem))
            copies.append(pltpu.async_copy(nll_hbm.at[row], nll_v.at[r], sem))
            copies.append(pltpu.async_copy(qa_hbm.at[row], qa_v.at[r], sem))
        for c in copies:
            c.wait()

        def combine_body(j, _):
            for u in range(2):
                sl = pl.ds((j * 2 + u) * LANES, LANES)
                for r in range(ROWS_PER_W):
                    c = (ng_v[r, sl] * 1.5
                         + (bt_v[r, sl] + nll_v[r, sl]) * 0.5) \
                        * (qa_v[r, sl] * 0.9 + 0.1)
                    comb_v[r, sl] = _vsort_desc(c)
            return 0

        lax.fori_loop(0, V // 2, combine_body, 0)

        def mirror_pass(w):
            def pass_body(j, _):
                for u in range(2):
                    jj = j * 2 + u
                    t = jj // w
                    i = jj % w
                    a = 2 * w * t + i
                    b = 2 * w * t + (2 * w - 1 - i)
                    sa = pl.ds(a * LANES, LANES)
                    sb = pl.ds(b * LANES, LANES)
                    for r in range(ROWS_PER_W):
                        va = comb_v[r, sa]
                        vb = lax.rev(comb_v[r, sb], (0,))
                        comb_v[r, sa] = jnp.maximum(va, vb)
                        comb_v[r, sb] = lax.rev(jnp.minimum(va, vb), (0,))
                return 0

            lax.fori_loop(0, HALF // 2, pass_body, 0)

        def inner_pass(d):
            def pass_body(j, _):
                for u in range(2):
                    jj = j * 2 + u
                    a = (jj // d) * (2 * d) + (jj % d)
                    b = a + d
                    sa = pl.ds(a * LANES, LANES)
                    sb = pl.ds(b * LANES, LANES)
                    for r in range(ROWS_PER_W):
                        va = comb_v[r, sa]
                        vb = comb_v[r, sb]
                        comb_v[r, sa] = jnp.maximum(va, vb)
                        comb_v[r, sb] = jnp.minimum(va, vb)
                return 0

            lax.fori_loop(0, HALF // 2, pass_body, 0)

        def vsort_pass():
            def pass_body(j, _):
                for u in range(2):
                    sl = pl.ds((j * 2 + u) * LANES, LANES)
                    for r in range(ROWS_PER_W):
                        comb_v[r, sl] = _vsort_desc(comb_v[r, sl])
                return 0

            lax.fori_loop(0, V // 2, pass_body, 0)

        w = 1
        while w <= V // 2:
            mirror_pass(w)
            d = w // 2
            while d >= 1:
                inner_pass(d)
                d //= 2
            vsort_pass()
            w *= 2

        for r in range(ROWS_PER_W):
            pltpu.sync_copy(comb_v.at[r], sorted_hbm.at[row0 + r])

    return pl.kernel(
        body,
        out_type=jax.ShapeDtypeStruct((B, N), jnp.float32),
        mesh=plsc.VectorSubcoreMesh(core_axis_name="c", subcore_axis_name="s"),
        compiler_params=pltpu.CompilerParams(needs_layout_passes=False),
        scratch_types=[
            pltpu.VMEM((ROWS_PER_W, N), jnp.float32),
            pltpu.VMEM((ROWS_PER_W, N), jnp.float32),
            pltpu.VMEM((ROWS_PER_W, N), jnp.float32),
            pltpu.VMEM((ROWS_PER_W, N), jnp.float32),
            pltpu.VMEM((ROWS_PER_W, N), jnp.float32),
            pltpu.SemaphoreType.DMA,
        ],
    )


def _tc_argmax_body(ng_ref, bt_ref, nll_ref, qa_ref, idx_ref):
    comb = (ng_ref[...] * 1.5 + (bt_ref[...] + nll_ref[...]) * 0.5) \
        * (qa_ref[...] * 0.9 + 0.1)
    m = jnp.max(comb, axis=1, keepdims=True)
    iota = lax.broadcasted_iota(jnp.int32, (B, N), 1)
    idx_ref[...] = jnp.min(jnp.where(comb == m, iota, N), axis=1,
                           keepdims=True)


@functools.lru_cache(maxsize=1)
def _build_tc_argmax():
    return pl.pallas_call(
        _tc_argmax_body,
        out_shape=jax.ShapeDtypeStruct((B, 1), jnp.int32),
    )


ROWS_PER_STEP = 8


def _tc_gather_body(idx_ref, *refs):
    cand_refs = refs[:ROWS_PER_STEP]
    out_ref, len_ref = refs[ROWS_PER_STEP:]
    g = pl.program_id(0)
    for k in range(ROWS_PER_STEP):
        sub = idx_ref[g * ROWS_PER_STEP + k, 0] % 8
        out_ref[pl.ds(k, 1), :] = cand_refs[k][0, pl.ds(sub, 1), :]
    vals = out_ref[...]
    len_ref[...] = jnp.sum((vals != PAD_ID).astype(jnp.int32), axis=1,
                           keepdims=True)


def _cand_spec(k):
    return pl.BlockSpec(
        (1, 8, L),
        lambda g, idx_ref, k=k: (g * ROWS_PER_STEP + k,
                                 idx_ref[g * ROWS_PER_STEP + k, 0] // 8, 0))


@functools.lru_cache(maxsize=1)
def _build_tc_gather():
    grid_spec = pltpu.PrefetchScalarGridSpec(
        num_scalar_prefetch=1,
        grid=(B // ROWS_PER_STEP,),
        in_specs=[_cand_spec(k) for k in range(ROWS_PER_STEP)],
        out_specs=[
            pl.BlockSpec((ROWS_PER_STEP, L), lambda g, idx_ref: (g, 0)),
            pl.BlockSpec((ROWS_PER_STEP, 1), lambda g, idx_ref: (g, 0)),
        ],
    )
    return pl.pallas_call(
        _tc_gather_body,
        grid_spec=grid_spec,
        out_shape=[
            jax.ShapeDtypeStruct((B, L), jnp.int32),
            jax.ShapeDtypeStruct((B, 1), jnp.int32),
        ],
    )


def kernel(candidates, lengths, scores, ngram_scores, backtrans_scores,
           qa_scores):
    del lengths
    sorted_scores = _build_sc_sort()(
        ngram_scores, backtrans_scores, scores, qa_scores)
    idx = _build_tc_argmax()(
        ngram_scores, backtrans_scores, scores, qa_scores)
    out, lens = _build_tc_gather()(
        idx, *([candidates] * ROWS_PER_STEP))
    return out, lens[:, 0], sorted_scores


# speedup vs baseline: 1.3839x; 1.3839x over previous
import functools

import jax
import jax.numpy as jnp
from jax import lax
from jax.experimental import pallas as pl
from jax.experimental.pallas import tpu as pltpu
from jax.experimental.pallas import tpu_sc as plsc

PAD_ID = 0
B, N, L = 64, 2048, 32
LANES = 16
V = N // LANES
HALF = V // 2
ROWS_PER_W = 2


def _vsort_desc(v):
    return lax.rev(jnp.sort(v), (0,))


@functools.lru_cache(maxsize=1)
def _build_sc_sort():
    info = plsc.get_sparse_core_info()
    nc = info.num_cores

    def body(ng_hbm, bt_hbm, nll_hbm, qa_hbm, sorted_hbm,
             ng_v, bt_v, nll_v, qa_v, comb_v, sem):
        wid = lax.axis_index("s") * nc + lax.axis_index("c")
        row0 = wid * ROWS_PER_W

        copies = []
        for r in range(ROWS_PER_W):
            row = row0 + r
            copies.append(pltpu.async_copy(ng_hbm.at[row], ng_v.at[r], sem))
            copies.append(pltpu.async_copy(bt_hbm.at[row], bt_v.at[r], sem))
            copies.append(pltpu.async_copy(nll_hbm.at[row], nll_v.at[r], sem))
            copies.append(pltpu.async_copy(qa_hbm.at[row], qa_v.at[r], sem))
        for c in copies:
            c.wait()

        def combine_body(j, _):
            for u in range(2):
                sl = pl.ds((j * 2 + u) * LANES, LANES)
                for r in range(ROWS_PER_W):
                    c = (ng_v[r, sl] * 1.5
                         + (bt_v[r, sl] + nll_v[r, sl]) * 0.5) \
                        * (qa_v[r, sl] * 0.9 + 0.1)
                    comb_v[r, sl] = _vsort_desc(c)
            return 0

        lax.fori_loop(0, V // 2, combine_body, 0)

        def mirror_pass(w):
            def pass_body(j, _):
                for u in range(2):
                    jj = j * 2 + u
                    t = jj // w
                    i = jj % w
                    a = 2 * w * t + i
                    b = 2 * w * t + (2 * w - 1 - i)
                    sa = pl.ds(a * LANES, LANES)
                    sb = pl.ds(b * LANES, LANES)
                    for r in range(ROWS_PER_W):
                        va = comb_v[r, sa]
                        vb = lax.rev(comb_v[r, sb], (0,))
                        comb_v[r, sa] = jnp.maximum(va, vb)
                        comb_v[r, sb] = lax.rev(jnp.minimum(va, vb), (0,))
                return 0

            lax.fori_loop(0, HALF // 2, pass_body, 0)

        def inner_pass(d):
            def pass_body(j, _):
                for u in range(2):
                    jj = j * 2 + u
                    a = (jj // d) * (2 * d) + (jj % d)
                    b = a + d
                    sa = pl.ds(a * LANES, LANES)
                    sb = pl.ds(b * LANES, LANES)
                    for r in range(ROWS_PER_W):
                        va = comb_v[r, sa]
                        vb = comb_v[r, sb]
                        comb_v[r, sa] = jnp.maximum(va, vb)
                        comb_v[r, sb] = jnp.minimum(va, vb)
                return 0

            lax.fori_loop(0, HALF // 2, pass_body, 0)

        def vsort_pass():
            def pass_body(j, _):
                for u in range(2):
                    sl = pl.ds((j * 2 + u) * LANES, LANES)
                    for r in range(ROWS_PER_W):
                        comb_v[r, sl] = _vsort_desc(comb_v[r, sl])
                return 0

            lax.fori_loop(0, V // 2, pass_body, 0)

        w = 1
        while w <= V // 2:
            mirror_pass(w)
            d = w // 2
            while d >= 1:
                inner_pass(d)
                d //= 2
            vsort_pass()
            w *= 2

        for r in range(ROWS_PER_W):
            pltpu.sync_copy(comb_v.at[r], sorted_hbm.at[row0 + r])

    return pl.kernel(
        body,
        out_type=jax.ShapeDtypeStruct((B, N), jnp.float32),
        mesh=plsc.VectorSubcoreMesh(core_axis_name="c", subcore_axis_name="s"),
        compiler_params=pltpu.CompilerParams(needs_layout_passes=False),
        scratch_types=[
            pltpu.VMEM((ROWS_PER_W, N), jnp.float32),
            pltpu.VMEM((ROWS_PER_W, N), jnp.float32),
            pltpu.VMEM((ROWS_PER_W, N), jnp.float32),
            pltpu.VMEM((ROWS_PER_W, N), jnp.float32),
            pltpu.VMEM((ROWS_PER_W, N), jnp.float32),
            pltpu.SemaphoreType.DMA,
        ],
    )


def _tc_argmax_body(ng_ref, bt_ref, nll_ref, qa_ref, idx_ref):
    comb = (ng_ref[...] * 1.5 + (bt_ref[...] + nll_ref[...]) * 0.5) \
        * (qa_ref[...] * 0.9 + 0.1)
    m = jnp.max(comb, axis=1, keepdims=True)
    iota = lax.broadcasted_iota(jnp.int32, (B, N), 1)
    idx_ref[...] = jnp.min(jnp.where(comb == m, iota, N), axis=1,
                           keepdims=True)


@functools.lru_cache(maxsize=1)
def _build_tc_argmax():
    return pl.pallas_call(
        _tc_argmax_body,
        out_shape=jax.ShapeDtypeStruct((B, 1), jnp.int32),
    )


LANE_BLK = 128


def _tc_gather_body(idx_ref, *refs):
    cand_refs = refs[:B]
    out_ref, len_ref = refs[B:]
    iota = lax.broadcasted_iota(jnp.int32, (L, LANE_BLK), 1)
    for b in range(B):
        m = idx_ref[b, 0] % LANE_BLK
        col = jnp.sum(jnp.where(iota == m, cand_refs[b][0], 0), axis=1)
        out_ref[pl.ds(b, 1), :] = col[None, :]
    vals = out_ref[...]
    len_ref[...] = jnp.sum((vals != PAD_ID).astype(jnp.int32), axis=1,
                           keepdims=True)


def _cand_spec(b):
    return pl.BlockSpec(
        (1, L, LANE_BLK),
        lambda g, idx_ref, b=b: (b, 0, idx_ref[b, 0] // LANE_BLK))


@functools.lru_cache(maxsize=1)
def _build_tc_gather():
    grid_spec = pltpu.PrefetchScalarGridSpec(
        num_scalar_prefetch=1,
        grid=(1,),
        in_specs=[_cand_spec(b) for b in range(B)],
        out_specs=[
            pl.BlockSpec((B, L), lambda g, idx_ref: (0, 0)),
            pl.BlockSpec((B, 1), lambda g, idx_ref: (0, 0)),
        ],
    )
    return pl.pallas_call(
        _tc_gather_body,
        grid_spec=grid_spec,
        out_shape=[
            jax.ShapeDtypeStruct((B, L), jnp.int32),
            jax.ShapeDtypeStruct((B, 1), jnp.int32),
        ],
    )


def kernel(candidates, lengths, scores, ngram_scores, backtrans_scores,
           qa_scores):
    del lengths
    sorted_scores = _build_sc_sort()(
        ngram_scores, backtrans_scores, scores, qa_scores)
    idx = _build_tc_argmax()(
        ngram_scores, backtrans_scores, scores, qa_scores)
    cand_t = jnp.transpose(candidates, (0, 2, 1))
    out, lens = _build_tc_gather()(idx, *([cand_t] * B))
    return out, lens[:, 0], sorted_scores


# speedup vs baseline: 1.8098x; 1.3077x over previous
import functools

import jax
import jax.numpy as jnp
from jax import lax
from jax.experimental import pallas as pl
from jax.experimental.pallas import tpu as pltpu
from jax.experimental.pallas import tpu_sc as plsc

PAD_ID = 0
B, N, L = 64, 2048, 32
LANES = 16
V = N // LANES
HALF = V // 2
ROWS_PER_W = 2


def _vsort_desc(v):
    return lax.rev(jnp.sort(v), (0,))


@functools.lru_cache(maxsize=1)
def _build_sc_sort():
    info = plsc.get_sparse_core_info()
    nc = info.num_cores

    def body(ng_hbm, bt_hbm, nll_hbm, qa_hbm, sorted_hbm,
             ng_v, bt_v, nll_v, qa_v, comb_v, sem):
        wid = lax.axis_index("s") * nc + lax.axis_index("c")
        row0 = wid * ROWS_PER_W

        copies = []
        for r in range(ROWS_PER_W):
            row = row0 + r
            copies.append(pltpu.async_copy(ng_hbm.at[row], ng_v.at[r], sem))
            copies.append(pltpu.async_copy(bt_hbm.at[row], bt_v.at[r], sem))
            copies.append(pltpu.async_copy(nll_hbm.at[row], nll_v.at[r], sem))
            copies.append(pltpu.async_copy(qa_hbm.at[row], qa_v.at[r], sem))
        for c in copies:
            c.wait()

        @plsc.parallel_loop(0, V, step=2, unroll=2)
        def _(j):
            for u in range(2):
                sl = pl.ds((j + u) * LANES, LANES)
                for r in range(ROWS_PER_W):
                    c = (ng_v[r, sl] * 1.5
                         + (bt_v[r, sl] + nll_v[r, sl]) * 0.5) \
                        * (qa_v[r, sl] * 0.9 + 0.1)
                    comb_v[r, sl] = _vsort_desc(c)

        def mirror_pass(w):
            @plsc.parallel_loop(0, HALF, step=2, unroll=2)
            def _(j):
                for u in range(2):
                    jj = j + u
                    t = jj // w
                    i = jj % w
                    a = 2 * w * t + i
                    b = 2 * w * t + (2 * w - 1 - i)
                    sa = pl.ds(a * LANES, LANES)
                    sb = pl.ds(b * LANES, LANES)
                    for r in range(ROWS_PER_W):
                        va = comb_v[r, sa]
                        vb = lax.rev(comb_v[r, sb], (0,))
                        comb_v[r, sa] = jnp.maximum(va, vb)
                        comb_v[r, sb] = lax.rev(jnp.minimum(va, vb), (0,))

        def inner_pass(d):
            @plsc.parallel_loop(0, HALF, step=2, unroll=2)
            def _(j):
                for u in range(2):
                    jj = j + u
                    a = (jj // d) * (2 * d) + (jj % d)
                    b = a + d
                    sa = pl.ds(a * LANES, LANES)
                    sb = pl.ds(b * LANES, LANES)
                    for r in range(ROWS_PER_W):
                        va = comb_v[r, sa]
                        vb = comb_v[r, sb]
                        comb_v[r, sa] = jnp.maximum(va, vb)
                        comb_v[r, sb] = jnp.minimum(va, vb)

        def vsort_pass():
            @plsc.parallel_loop(0, V, step=2, unroll=2)
            def _(j):
                for u in range(2):
                    sl = pl.ds((j + u) * LANES, LANES)
                    for r in range(ROWS_PER_W):
                        comb_v[r, sl] = _vsort_desc(comb_v[r, sl])

        w = 1
        while w <= V // 2:
            mirror_pass(w)
            d = w // 2
            while d >= 1:
                inner_pass(d)
                d //= 2
            vsort_pass()
            w *= 2

        for r in range(ROWS_PER_W):
            pltpu.sync_copy(comb_v.at[r], sorted_hbm.at[row0 + r])

    return pl.kernel(
        body,
        out_type=jax.ShapeDtypeStruct((B, N), jnp.float32),
        mesh=plsc.VectorSubcoreMesh(core_axis_name="c", subcore_axis_name="s"),
        compiler_params=pltpu.CompilerParams(needs_layout_passes=False),
        scratch_types=[
            pltpu.VMEM((ROWS_PER_W, N), jnp.float32),
            pltpu.VMEM((ROWS_PER_W, N), jnp.float32),
            pltpu.VMEM((ROWS_PER_W, N), jnp.float32),
            pltpu.VMEM((ROWS_PER_W, N), jnp.float32),
            pltpu.VMEM((ROWS_PER_W, N), jnp.float32),
            pltpu.SemaphoreType.DMA,
        ],
    )


def _tc_argmax_body(ng_ref, bt_ref, nll_ref, qa_ref, idx_ref):
    comb = (ng_ref[...] * 1.5 + (bt_ref[...] + nll_ref[...]) * 0.5) \
        * (qa_ref[...] * 0.9 + 0.1)
    m = jnp.max(comb, axis=1, keepdims=True)
    iota = lax.broadcasted_iota(jnp.int32, (B, N), 1)
    idx_ref[...] = jnp.min(jnp.where(comb == m, iota, N), axis=1,
                           keepdims=True)


@functools.lru_cache(maxsize=1)
def _build_tc_argmax():
    return pl.pallas_call(
        _tc_argmax_body,
        out_shape=jax.ShapeDtypeStruct((B, 1), jnp.int32),
    )


LANE_BLK = 128


def _tc_gather_body(idx_ref, *refs):
    cand_refs = refs[:B]
    out_ref, len_ref = refs[B:]
    iota = lax.broadcasted_iota(jnp.int32, (L, LANE_BLK), 1)
    for b in range(B):
        m = idx_ref[b, 0] % LANE_BLK
        col = jnp.sum(jnp.where(iota == m, cand_refs[b][0], 0), axis=1)
        out_ref[pl.ds(b, 1), :] = col[None, :]
    vals = out_ref[...]
    len_ref[...] = jnp.sum((vals != PAD_ID).astype(jnp.int32), axis=1,
                           keepdims=True)


def _cand_spec(b):
    return pl.BlockSpec(
        (1, L, LANE_BLK),
        lambda g, idx_ref, b=b: (b, 0, idx_ref[b, 0] // LANE_BLK))


@functools.lru_cache(maxsize=1)
def _build_tc_gather():
    grid_spec = pltpu.PrefetchScalarGridSpec(
        num_scalar_prefetch=1,
        grid=(1,),
        in_specs=[_cand_spec(b) for b in range(B)],
        out_specs=[
            pl.BlockSpec((B, L), lambda g, idx_ref: (0, 0)),
            pl.BlockSpec((B, 1), lambda g, idx_ref: (0, 0)),
        ],
    )
    return pl.pallas_call(
        _tc_gather_body,
        grid_spec=grid_spec,
        out_shape=[
            jax.ShapeDtypeStruct((B, L), jnp.int32),
            jax.ShapeDtypeStruct((B, 1), jnp.int32),
        ],
    )


def kernel(candidates, lengths, scores, ngram_scores, backtrans_scores,
           qa_scores):
    del lengths
    sorted_scores = _build_sc_sort()(
        ngram_scores, backtrans_scores, scores, qa_scores)
    idx = _build_tc_argmax()(
        ngram_scores, backtrans_scores, scores, qa_scores)
    cand_t = jnp.transpose(candidates, (0, 2, 1))
    out, lens = _build_tc_gather()(idx, *([cand_t] * B))
    return out, lens[:, 0], sorted_scores


# speedup vs baseline: 1.8908x; 1.0448x over previous
import functools

import jax
import jax.numpy as jnp
from jax import lax
from jax.experimental import pallas as pl
from jax.experimental.pallas import tpu as pltpu
from jax.experimental.pallas import tpu_sc as plsc

PAD_ID = 0
B, N, L = 64, 2048, 32
LANES = 16
V = N // LANES
HALF = V // 2
ROWS_PER_W = 2


def _vsort_desc(v):
    return lax.rev(jnp.sort(v), (0,))


@functools.lru_cache(maxsize=1)
def _build_sc_sort():
    info = plsc.get_sparse_core_info()
    nc = info.num_cores

    def body(ng_hbm, bt_hbm, nll_hbm, qa_hbm, sorted_hbm,
             ng_v, bt_v, nll_v, qa_v, comb_v, sem):
        wid = lax.axis_index("s") * nc + lax.axis_index("c")
        row0 = wid * ROWS_PER_W

        copies = []
        for r in range(ROWS_PER_W):
            row = row0 + r
            copies.append(pltpu.async_copy(ng_hbm.at[row], ng_v.at[r], sem))
            copies.append(pltpu.async_copy(bt_hbm.at[row], bt_v.at[r], sem))
            copies.append(pltpu.async_copy(nll_hbm.at[row], nll_v.at[r], sem))
            copies.append(pltpu.async_copy(qa_hbm.at[row], qa_v.at[r], sem))
        for c in copies:
            c.wait()

        @plsc.parallel_loop(0, V, step=2, unroll=4)
        def _(j):
            for u in range(2):
                sl = pl.ds((j + u) * LANES, LANES)
                for r in range(ROWS_PER_W):
                    c = (ng_v[r, sl] * 1.5
                         + (bt_v[r, sl] + nll_v[r, sl]) * 0.5) \
                        * (qa_v[r, sl] * 0.9 + 0.1)
                    comb_v[r, sl] = _vsort_desc(c)

        def mirror_pass(w):
            @plsc.parallel_loop(0, HALF, step=2, unroll=4)
            def _(j):
                for u in range(2):
                    jj = j + u
                    t = jj // w
                    i = jj % w
                    a = 2 * w * t + i
                    b = 2 * w * t + (2 * w - 1 - i)
                    sa = pl.ds(a * LANES, LANES)
                    sb = pl.ds(b * LANES, LANES)
                    for r in range(ROWS_PER_W):
                        va = comb_v[r, sa]
                        vb = lax.rev(comb_v[r, sb], (0,))
                        comb_v[r, sa] = jnp.maximum(va, vb)
                        comb_v[r, sb] = lax.rev(jnp.minimum(va, vb), (0,))

        def inner_pass(d):
            @plsc.parallel_loop(0, HALF, step=2, unroll=4)
            def _(j):
                for u in range(2):
                    jj = j + u
                    a = (jj // d) * (2 * d) + (jj % d)
                    b = a + d
                    sa = pl.ds(a * LANES, LANES)
                    sb = pl.ds(b * LANES, LANES)
                    for r in range(ROWS_PER_W):
                        va = comb_v[r, sa]
                        vb = comb_v[r, sb]
                        comb_v[r, sa] = jnp.maximum(va, vb)
                        comb_v[r, sb] = jnp.minimum(va, vb)

        def vsort_pass():
            @plsc.parallel_loop(0, V, step=2, unroll=4)
            def _(j):
                for u in range(2):
                    sl = pl.ds((j + u) * LANES, LANES)
                    for r in range(ROWS_PER_W):
                        comb_v[r, sl] = _vsort_desc(comb_v[r, sl])

        w = 1
        while w <= V // 2:
            mirror_pass(w)
            d = w // 2
            while d >= 1:
                inner_pass(d)
                d //= 2
            vsort_pass()
            w *= 2

        for r in range(ROWS_PER_W):
            pltpu.sync_copy(comb_v.at[r], sorted_hbm.at[row0 + r])

    return pl.kernel(
        body,
        out_type=jax.ShapeDtypeStruct((B, N), jnp.float32),
        mesh=plsc.VectorSubcoreMesh(core_axis_name="c", subcore_axis_name="s"),
        compiler_params=pltpu.CompilerParams(needs_layout_passes=False),
        scratch_types=[
            pltpu.VMEM((ROWS_PER_W, N), jnp.float32),
            pltpu.VMEM((ROWS_PER_W, N), jnp.float32),
            pltpu.VMEM((ROWS_PER_W, N), jnp.float32),
            pltpu.VMEM((ROWS_PER_W, N), jnp.float32),
            pltpu.VMEM((ROWS_PER_W, N), jnp.float32),
            pltpu.SemaphoreType.DMA,
        ],
    )


def _tc_argmax_body(ng_ref, bt_ref, nll_ref, qa_ref, idx_ref):
    comb = (ng_ref[...] * 1.5 + (bt_ref[...] + nll_ref[...]) * 0.5) \
        * (qa_ref[...] * 0.9 + 0.1)
    m = jnp.max(comb, axis=1, keepdims=True)
    iota = lax.broadcasted_iota(jnp.int32, (B, N), 1)
    idx_ref[...] = jnp.min(jnp.where(comb == m, iota, N), axis=1,
                           keepdims=True)


@functools.lru_cache(maxsize=1)
def _build_tc_argmax():
    return pl.pallas_call(
        _tc_argmax_body,
        out_shape=jax.ShapeDtypeStruct((B, 1), jnp.int32),
    )


LANE_BLK = 128


def _tc_gather_body(idx_ref, *refs):
    cand_refs = refs[:B]
    out_ref, len_ref = refs[B:]
    iota = lax.broadcasted_iota(jnp.int32, (L, LANE_BLK), 1)
    for b in range(B):
        m = idx_ref[b, 0] % LANE_BLK
        col = jnp.sum(jnp.where(iota == m, cand_refs[b][0], 0), axis=1)
        out_ref[pl.ds(b, 1), :] = col[None, :]
    vals = out_ref[...]
    len_ref[...] = jnp.sum((vals != PAD_ID).astype(jnp.int32), axis=1,
                           keepdims=True)


def _cand_spec(b):
    return pl.BlockSpec(
        (1, L, LANE_BLK),
        lambda g, idx_ref, b=b: (b, 0, idx_ref[b, 0] // LANE_BLK))


@functools.lru_cache(maxsize=1)
def _build_tc_gather():
    grid_spec = pltpu.PrefetchScalarGridSpec(
        num_scalar_prefetch=1,
        grid=(1,),
        in_specs=[_cand_spec(b) for b in range(B)],
        out_specs=[
            pl.BlockSpec((B, L), lambda g, idx_ref: (0, 0)),
            pl.BlockSpec((B, 1), lambda g, idx_ref: (0, 0)),
        ],
    )
    return pl.pallas_call(
        _tc_gather_body,
        grid_spec=grid_spec,
        out_shape=[
            jax.ShapeDtypeStruct((B, L), jnp.int32),
            jax.ShapeDtypeStruct((B, 1), jnp.int32),
        ],
    )


def kernel(candidates, lengths, scores, ngram_scores, backtrans_scores,
           qa_scores):
    del lengths
    sorted_scores = _build_sc_sort()(
        ngram_scores, backtrans_scores, scores, qa_scores)
    idx = _build_tc_argmax()(
        ngram_scores, backtrans_scores, scores, qa_scores)
    cand_t = jnp.transpose(candidates, (0, 2, 1))
    out, lens = _build_tc_gather()(idx, *([cand_t] * B))
    return out, lens[:, 0], sorted_scores
